# baseline (speedup 1.0000x reference)
import jax
import jax.numpy as jnp
from jax import lax
from jax.experimental import pallas as pl
from jax.experimental.pallas import tpu as pltpu
from jax.experimental.pallas import tpu_sc as plsc

B = 16384
C = 1000
RTC = 992
NC = 2
NS = 16
NW = NC * NS
BPW = B // NW
LANES = 16
NGRP = BPW // LANES
CROWS = 64
_CHUNKS = []
_r = RTC
while _r < C:
    _CHUNKS.append((_r, min(CROWS, C - _r)))
    _r += CROWS

CB = 2048
NBLK = B // CB


def _sc_body(cond_hbm, out_hbm, idx_v, buf0, buf1, sem0, sem1):
    wid = lax.axis_index("s") * NC + lax.axis_index("c")
    bbase = wid * BPW
    pltpu.sync_copy(cond_hbm.at[pl.ds(bbase, BPW)], idx_v)

    zeros = jnp.zeros((LANES,), jnp.float32)
    ones = jnp.ones((LANES,), jnp.float32)
    lane = lax.iota(jnp.int32, LANES)

    def zfill(i, carry):
        for s in range(CROWS):
            buf0[s, pl.ds(i * LANES, LANES)] = zeros
            buf1[s, pl.ds(i * LANES, LANES)] = zeros
        return carry
    lax.fori_loop(0, NGRP, zfill, 0)

    bufs = (buf0, buf1)
    sems = (sem0, sem1)

    def patch(buf, row0, nrows, x):
        def grp(g, carry):
            cond16 = idx_v[pl.ds(g * LANES, LANES)]
            rows = cond16 - row0
            mask = (rows >= 0) & (rows < nrows)
            rows = jnp.where(mask, rows, 0)
            cols = g * LANES + lane
            plsc.store_scatter(buf, [rows, cols], x, mask=mask)
            return carry
        lax.fori_loop(0, NGRP, grp, 0)

    def dma(k):
        row0, nrows = _CHUNKS[k]
        src = bufs[k % 2]
        if nrows != CROWS:
            src = src.at[pl.ds(0, nrows)]
        return pltpu.make_async_copy(
            src, out_hbm.at[pl.ds(row0, nrows), pl.ds(bbase, BPW)],
            sems[k % 2])

    for k, (row0, nrows) in enumerate(_CHUNKS):
        b = k % 2
        if k >= 2:
            dma(k - 2).wait()
            prow0, pnrows = _CHUNKS[k - 2]
            patch(bufs[b], prow0, pnrows, zeros)
        patch(bufs[b], row0, nrows, ones)
        dma(k).start()

    nch = len(_CHUNKS)
    if nch >= 2:
        dma(nch - 2).wait()
    dma(nch - 1).wait()


def _make_tc_body(ntc):
    bpc = NBLK // ntc

    def _tc_body(cond_hbm, out_hbm, cond_v, tbuf0, tbuf1, tsem0, tsem1):
        tid = lax.axis_index("t")
        pltpu.sync_copy(cond_hbm, cond_v)

        tbufs = (tbuf0, tbuf1)
        tsems = (tsem0, tsem1)

        def col0(i):
            return (tid * bpc + i) * CB

        def dma(i):
            return pltpu.make_async_copy(
                tbufs[i % 2],
                out_hbm.at[pl.ds(0, RTC), pl.ds(col0(i), CB)],
                tsems[i % 2])

        ridx = lax.broadcasted_iota(jnp.int32, (RTC, CB), 0)
        for i in range(bpc):
            if i >= 2:
                dma(i - 2).wait()
            condb = cond_v[pl.ds(col0(i), CB)]
            tbufs[i % 2][...] = (ridx == condb[None, :]).astype(jnp.float32)
            dma(i).start()
        if bpc >= 2:
            dma(bpc - 2).wait()
        dma(bpc - 1).wait()

    return _tc_body


def kernel(cond):
    sc_mesh = plsc.VectorSubcoreMesh(
        core_axis_name="c", subcore_axis_name="s", num_cores=NC
    )
    tc_mesh = pltpu.create_tensorcore_mesh("t", num_cores=2)
    ntc = int(tc_mesh.devices.size)

    out_ref = jax.empty_ref(
        jax.ShapeDtypeStruct((C, B), jnp.float32)
    )

    tc_fill = pl.kernel(
        _make_tc_body(ntc),
        out_type=(),
        mesh=tc_mesh,
        scratch_types=[
            pltpu.VMEM((B,), jnp.int32),
            pltpu.VMEM((RTC, CB), jnp.float32),
            pltpu.VMEM((RTC, CB), jnp.float32),
            pltpu.SemaphoreType.DMA,
            pltpu.SemaphoreType.DMA,
        ],
    )
    sc_fill = pl.kernel(
        _sc_body,
        out_type=(),
        mesh=sc_mesh,
        compiler_params=pltpu.CompilerParams(
            needs_layout_passes=False, use_tc_tiling_on_sc=True
        ),
        scratch_types=[
            pltpu.VMEM((BPW,), jnp.int32),
            pltpu.VMEM((CROWS, BPW), jnp.float32),
            pltpu.VMEM((CROWS, BPW), jnp.float32),
            pltpu.SemaphoreType.DMA,
            pltpu.SemaphoreType.DMA,
        ],
    )

    tc_fill(cond, out_ref)
    sc_fill(cond, out_ref)
    out2d = out_ref[...]
    return out2d.T.reshape(B, 1, C)


# speedup vs baseline: 1.0612x; 1.0612x over previous
import jax
import jax.numpy as jnp
from jax import lax
from jax.experimental import pallas as pl
from jax.experimental.pallas import tpu as pltpu
from jax.experimental.pallas import tpu_sc as plsc

B = 16384
C = 1000
RTC = 504
NC = 2
NS = 16
NW = NC * NS
BPW = B // NW
LANES = 16
NGRP = BPW // LANES
CROWS = 64
_CHUNKS = []
_r = RTC
while _r < C:
    _CHUNKS.append((_r, min(CROWS, C - _r)))
    _r += CROWS

CB = 2048
NBLK = B // CB


def _sc_body(cond_hbm, out_hbm, idx_v, buf0, buf1, sem0, sem1):
    wid = lax.axis_index("s") * NC + lax.axis_index("c")
    bbase = wid * BPW
    pltpu.sync_copy(cond_hbm.at[pl.ds(bbase, BPW)], idx_v)

    zeros = jnp.zeros((LANES,), jnp.float32)
    ones = jnp.ones((LANES,), jnp.float32)
    lane = lax.iota(jnp.int32, LANES)

    def zfill(i, carry):
        for s in range(CROWS):
            buf0[s, pl.ds(i * LANES, LANES)] = zeros
            buf1[s, pl.ds(i * LANES, LANES)] = zeros
        return carry
    lax.fori_loop(0, NGRP, zfill, 0)

    bufs = (buf0, buf1)
    sems = (sem0, sem1)

    def patch(buf, row0, nrows, x):
        def grp(g, carry):
            cond16 = idx_v[pl.ds(g * LANES, LANES)]
            rows = cond16 - row0
            mask = (rows >= 0) & (rows < nrows)
            rows = jnp.where(mask, rows, 0)
            cols = g * LANES + lane
            plsc.store_scatter(buf, [rows, cols], x, mask=mask)
            return carry
        lax.fori_loop(0, NGRP, grp, 0)

    def dma(k):
        row0, nrows = _CHUNKS[k]
        src = bufs[k % 2]
        if nrows != CROWS:
            src = src.at[pl.ds(0, nrows)]
        return pltpu.make_async_copy(
            src, out_hbm.at[pl.ds(row0, nrows), pl.ds(bbase, BPW)],
            sems[k % 2])

    for k, (row0, nrows) in enumerate(_CHUNKS):
        b = k % 2
        if k >= 2:
            dma(k - 2).wait()
            prow0, pnrows = _CHUNKS[k - 2]
            patch(bufs[b], prow0, pnrows, zeros)
        patch(bufs[b], row0, nrows, ones)
        dma(k).start()

    nch = len(_CHUNKS)
    if nch >= 2:
        dma(nch - 2).wait()
    dma(nch - 1).wait()


def _make_tc_body(ntc):
    bpc = NBLK // ntc

    def _tc_body(cond_hbm, out_hbm, cond_v, tbuf0, tbuf1, tsem0, tsem1):
        tid = lax.axis_index("t")
        pltpu.sync_copy(cond_hbm, cond_v)

        tbufs = (tbuf0, tbuf1)
        tsems = (tsem0, tsem1)

        def col0(i):
            return (tid * bpc + i) * CB

        def dma(i):
            return pltpu.make_async_copy(
                tbufs[i % 2],
                out_hbm.at[pl.ds(0, RTC), pl.ds(col0(i), CB)],
                tsems[i % 2])

        ridx = lax.broadcasted_iota(jnp.int32, (RTC, CB), 0)
        for i in range(bpc):
            if i >= 2:
                dma(i - 2).wait()
            condb = cond_v[pl.ds(col0(i), CB)]
            tbufs[i % 2][...] = (ridx == condb[None, :]).astype(jnp.float32)
            dma(i).start()
        if bpc >= 2:
            dma(bpc - 2).wait()
        dma(bpc - 1).wait()

    return _tc_body


def kernel(cond):
    sc_mesh = plsc.VectorSubcoreMesh(
        core_axis_name="c", subcore_axis_name="s", num_cores=NC
    )
    tc_mesh = pltpu.create_tensorcore_mesh("t")
    ntc = int(tc_mesh.devices.size)
    if NBLK % ntc != 0:
        tc_mesh = pltpu.create_tensorcore_mesh("t", num_cores=1)
        ntc = 1

    out_ref = jax.empty_ref(
        jax.ShapeDtypeStruct((C, B), jnp.float32)
    )

    tc_fill = pl.kernel(
        _make_tc_body(ntc),
        out_type=(),
        mesh=tc_mesh,
        scratch_types=[
            pltpu.VMEM((B,), jnp.int32),
            pltpu.VMEM((RTC, CB), jnp.float32),
            pltpu.VMEM((RTC, CB), jnp.float32),
            pltpu.SemaphoreType.DMA,
            pltpu.SemaphoreType.DMA,
        ],
    )
    sc_fill = pl.kernel(
        _sc_body,
        out_type=(),
        mesh=sc_mesh,
        compiler_params=pltpu.CompilerParams(
            needs_layout_passes=False, use_tc_tiling_on_sc=True
        ),
        scratch_types=[
            pltpu.VMEM((BPW,), jnp.int32),
            pltpu.VMEM((CROWS, BPW), jnp.float32),
            pltpu.VMEM((CROWS, BPW), jnp.float32),
            pltpu.SemaphoreType.DMA,
            pltpu.SemaphoreType.DMA,
        ],
    )

    tc_fill(cond, out_ref)
    sc_fill(cond, out_ref)
    out2d = out_ref[...]
    return out2d.T.reshape(B, 1, C)
